# baseline (speedup 1.0000x reference)
import functools

import jax
import jax.numpy as jnp
from jax import lax
from jax.experimental import pallas as pl
from jax.experimental.pallas import tpu as pltpu
from jax.experimental.pallas import tpu_sc as plsc

BATCH = 16384
DIM = 64
NC = 2
NS = 16
NW = NC * NS
BPW = BATCH // NW
PASS = 128
NPASS = BPW // PASS
PGROUPS = PASS // 16
TXB = 512
MU = ((1000000 // 2 + TXB - 1) // TXB) * TXB
MI = ((100000 // 2 + TXB - 1) // TXB) * TXB


def _tx_body(ta_ref, tb_ref, o_ref):
    o_ref[...] = jnp.concatenate(
        [ta_ref[...].T, tb_ref[...].T], axis=1)


def _tc_pack(t, m):
    nblk = m // TXB
    return pl.pallas_call(
        _tx_body,
        grid=(nblk,),
        in_specs=[
            pl.BlockSpec((DIM, TXB), lambda i: (0, i)),
            pl.BlockSpec((DIM, TXB), lambda i: (0, i + nblk)),
        ],
        out_specs=pl.BlockSpec((TXB, 2 * DIM), lambda i: (i, 0)),
        out_shape=jax.ShapeDtypeStruct((m, 2 * DIM), jnp.float32),
    )(t, t)


def _sc_body(uidx_hbm, iidx_hbm, gm_hbm, umean_hbm, imean_hbm,
             upack_hbm, ipack_hbm, out_hbm,
             uidx_v, iidx_v, updx_v, ipdx_v, urows_v, irows_v,
             ubias_v, ibias_v, gm_v, out_v, sems, bsem):
    wid = lax.axis_index("s") * NC + lax.axis_index("c")
    base = wid * BPW

    pltpu.sync_copy(uidx_hbm.at[wid], uidx_v)
    pltpu.sync_copy(iidx_hbm.at[wid], iidx_v)
    pltpu.sync_copy(gm_hbm, gm_v)

    bias_copies = []
    for c in range(NPASS):
        dst = pl.ds(c * PASS, PASS)
        bias_copies.append(pltpu.async_copy(
            umean_hbm.at[uidx_v.at[c]], ubias_v.at[dst], bsem))
        bias_copies.append(pltpu.async_copy(
            imean_hbm.at[iidx_v.at[c]], ibias_v.at[dst], bsem))

    for p in range(NPASS):
        for k in range(PASS // 16):
            sl = pl.ds(k * 16, 16)
            uv = uidx_v[p, sl]
            iv = iidx_v[p, sl]
            updx_v[p, sl] = uv - (uv >= MU).astype(jnp.int32) * MU
            ipdx_v[p, sl] = iv - (iv >= MI).astype(jnp.int32) * MI

    copies = [None] * NPASS

    def fire(p):
        buf = p & 1
        sem = sems.at[buf]
        copies[p] = (
            pltpu.async_copy(upack_hbm.at[updx_v.at[p]],
                             urows_v.at[buf], sem),
            pltpu.async_copy(ipack_hbm.at[ipdx_v.at[p]],
                             irows_v.at[buf], sem),
        )

    gmv = gm_v[...]
    lane = jnp.arange(16, dtype=jnp.int32)

    def compute(p):
        buf = p & 1
        urows = urows_v.at[buf]
        irows = irows_v.at[buf]
        for g in range(PGROUPS):
            row0 = g * 16
            rows = row0 + lane
            b0 = p * PASS + row0
            sl = pl.ds(b0, 16)
            psl = pl.ds(row0, 16)
            ucol0 = (uidx_v[p, psl] >= MU).astype(jnp.int32) * DIM
            icol0 = (iidx_v[p, psl] >= MI).astype(jnp.int32) * DIM
            acc = gmv + ubias_v[sl] + ibias_v[sl]
            for d in range(DIM):
                u = plsc.load_gather(urows, [rows, ucol0 + d])
                v = plsc.load_gather(irows, [rows, icol0 + d])
                acc = acc + u * v
            out_v[sl] = acc

    fire(0)
    for cp in bias_copies:
        cp.wait()
    for p in range(NPASS):
        if p + 1 < NPASS:
            fire(p + 1)
        for cp in copies[p]:
            cp.wait()
        compute(p)

    pltpu.sync_copy(out_v, out_hbm.at[pl.ds(base, BPW)])


def kernel(user_mapped, item_mapped, global_mean, user_mean, item_mean,
           user_implicit, item_implicit):
    uidx3 = user_mapped.reshape(NW, NPASS, PASS)
    iidx3 = item_mapped.reshape(NW, NPASS, PASS)
    upack = _tc_pack(user_implicit.T, MU)
    ipack = _tc_pack(item_implicit.T, MI)
    gm_vec = jnp.broadcast_to(global_mean.astype(jnp.float32), (16,))

    mesh = plsc.VectorSubcoreMesh(
        core_axis_name="c", subcore_axis_name="s",
        num_cores=NC, num_subcores=NS)

    run = pl.kernel(
        _sc_body,
        out_type=jax.ShapeDtypeStruct((BATCH,), jnp.float32),
        mesh=mesh,
        compiler_params=pltpu.CompilerParams(needs_layout_passes=False),
        scratch_types=[
            pltpu.VMEM((NPASS, PASS), jnp.int32),
            pltpu.VMEM((NPASS, PASS), jnp.int32),
            pltpu.VMEM((NPASS, PASS), jnp.int32),
            pltpu.VMEM((NPASS, PASS), jnp.int32),
            pltpu.VMEM((2, PASS, 2 * DIM), jnp.float32),
            pltpu.VMEM((2, PASS, 2 * DIM), jnp.float32),
            pltpu.VMEM((BPW,), jnp.float32),
            pltpu.VMEM((BPW,), jnp.float32),
            pltpu.VMEM((16,), jnp.float32),
            pltpu.VMEM((BPW,), jnp.float32),
            pltpu.SemaphoreType.DMA((2,)),
            pltpu.SemaphoreType.DMA,
        ],
    )
    return run(uidx3, iidx3, gm_vec, user_mean, item_mean, upack, ipack)


# speedup vs baseline: 1.8458x; 1.8458x over previous
import functools

import jax
import jax.numpy as jnp
from jax import lax
from jax.experimental import pallas as pl
from jax.experimental.pallas import tpu as pltpu
from jax.experimental.pallas import tpu_sc as plsc

BATCH = 16384
DIM = 64
NC = 2
NS = 16
NW = NC * NS
BPW = BATCH // NW
PASS = 128
NPASS = BPW // PASS
PGROUPS = PASS // 16


def _sc_body(uidx_hbm, iidx_hbm, gm_hbm, umean_hbm, imean_hbm,
             uimp_hbm, iimp_hbm, out_hbm,
             uidx_v, iidx_v, urows_v, irows_v,
             ubias_v, ibias_v, gm_v, out_v, sems, bsem):
    wid = lax.axis_index("s") * NC + lax.axis_index("c")
    base = wid * BPW

    pltpu.sync_copy(uidx_hbm.at[wid], uidx_v)
    pltpu.sync_copy(iidx_hbm.at[wid], iidx_v)
    pltpu.sync_copy(gm_hbm, gm_v)

    bias_copies = []
    for c in range(NPASS):
        dst = pl.ds(c * PASS, PASS)
        bias_copies.append(pltpu.async_copy(
            umean_hbm.at[uidx_v.at[c]], ubias_v.at[dst], bsem))
        bias_copies.append(pltpu.async_copy(
            imean_hbm.at[iidx_v.at[c]], ibias_v.at[dst], bsem))

    def fire(p):
        buf = p & 1
        sem = sems.at[buf]

        def fetch(jv, carry):
            j16 = jv * 16
            uvec = uidx_v[p, pl.ds(j16, 16)]
            ivec = iidx_v[p, pl.ds(j16, 16)]
            for k in range(16):
                pltpu.async_copy(
                    uimp_hbm.at[uvec[k]], urows_v.at[buf, j16 + k], sem)
                pltpu.async_copy(
                    iimp_hbm.at[ivec[k]], irows_v.at[buf, j16 + k], sem)
            return carry

        lax.fori_loop(0, PASS // 16, fetch, 0)

    def drain(p):
        buf = p & 1
        sem = sems.at[buf]
        pltpu.make_async_copy(
            uimp_hbm.at[pl.ds(0, PASS)], urows_v.at[buf], sem).wait()
        pltpu.make_async_copy(
            iimp_hbm.at[pl.ds(0, PASS)], irows_v.at[buf], sem).wait()

    gmv = gm_v[...]
    lane = jnp.arange(16, dtype=jnp.int32)

    def compute(p):
        buf = p & 1
        urows = urows_v.at[buf]
        irows = irows_v.at[buf]
        for g in range(PGROUPS):
            row0 = g * 16
            rows = row0 + lane
            b0 = p * PASS + row0
            sl = pl.ds(b0, 16)
            acc = gmv + ubias_v[sl] + ibias_v[sl]
            for d in range(DIM):
                cols = jnp.full((16,), d, dtype=jnp.int32)
                u = plsc.load_gather(urows, [rows, cols])
                v = plsc.load_gather(irows, [rows, cols])
                acc = acc + u * v
            out_v[sl] = acc

    fire(0)
    for cp in bias_copies:
        cp.wait()
    for p in range(NPASS):
        if p + 1 < NPASS:
            fire(p + 1)
        drain(p)
        compute(p)

    pltpu.sync_copy(out_v, out_hbm.at[pl.ds(base, BPW)])


def kernel(user_mapped, item_mapped, global_mean, user_mean, item_mean,
           user_implicit, item_implicit):
    uidx3 = user_mapped.reshape(NW, NPASS, PASS)
    iidx3 = item_mapped.reshape(NW, NPASS, PASS)
    gm_vec = jnp.broadcast_to(global_mean.astype(jnp.float32), (16,))

    mesh = plsc.VectorSubcoreMesh(
        core_axis_name="c", subcore_axis_name="s",
        num_cores=NC, num_subcores=NS)

    run = pl.kernel(
        _sc_body,
        out_type=jax.ShapeDtypeStruct((BATCH,), jnp.float32),
        mesh=mesh,
        compiler_params=pltpu.CompilerParams(
            needs_layout_passes=False, use_tc_tiling_on_sc=True),
        scratch_types=[
            pltpu.VMEM((NPASS, PASS), jnp.int32),
            pltpu.VMEM((NPASS, PASS), jnp.int32),
            pltpu.VMEM((2, PASS, DIM), jnp.float32),
            pltpu.VMEM((2, PASS, DIM), jnp.float32),
            pltpu.VMEM((BPW,), jnp.float32),
            pltpu.VMEM((BPW,), jnp.float32),
            pltpu.VMEM((16,), jnp.float32),
            pltpu.VMEM((BPW,), jnp.float32),
            pltpu.SemaphoreType.DMA((2,)),
            pltpu.SemaphoreType.DMA,
        ],
    )
    return run(uidx3, iidx3, gm_vec, user_mean, item_mean,
               user_implicit, item_implicit)
